# Initial kernel scaffold; baseline (speedup 1.0000x reference)
import jax
import jax.numpy as jnp
from jax.experimental import pallas as pl


def kernel(val, obs_idx, feat_idx, W_val, b_val, emb_obs, emb_feat):
    raise NotImplementedError("write your pallas kernel here")


# speedup vs baseline: 4.3536x; 4.3536x over previous
import functools

import jax
import jax.numpy as jnp
from jax import lax
from jax.experimental import pallas as pl
from jax.experimental.pallas import tpu as pltpu
from jax.experimental.pallas import tpu_sc as plsc

D_MODEL = 64
N_OBS = 200
N_FEAT = 128
NJ = D_MODEL // 16

NUM_CORES = 2
NUM_SUBCORES = 16
NW = NUM_CORES * NUM_SUBCORES

BLK = 640


@functools.lru_cache(maxsize=None)
def _build(T: int):
    per_w = T // NW
    steps = per_w // BLK
    assert per_w % BLK == 0 and steps % 2 == 0

    mesh = plsc.VectorSubcoreMesh(
        core_axis_name="c", subcore_axis_name="s",
        num_cores=NUM_CORES, num_subcores=NUM_SUBCORES)

    @functools.partial(
        pl.kernel,
        out_type=jax.ShapeDtypeStruct((T * D_MODEL,), jnp.float32),
        mesh=mesh,
        scratch_types=[
            pltpu.VMEM((N_OBS * D_MODEL,), jnp.float32),
            pltpu.VMEM((N_FEAT * D_MODEL,), jnp.float32),
            pltpu.VMEM((D_MODEL,), jnp.float32),
            pltpu.VMEM((D_MODEL,), jnp.float32),
            pltpu.VMEM((2 * BLK,), jnp.int32),
            pltpu.VMEM((2 * BLK,), jnp.int32),
            pltpu.VMEM((2 * BLK,), jnp.float32),
            pltpu.VMEM((2 * BLK * D_MODEL,), jnp.float32),
            pltpu.SemaphoreType.DMA,
            pltpu.SemaphoreType.DMA,
            pltpu.SemaphoreType.DMA,
            pltpu.SemaphoreType.DMA,
        ],
    )
    def embed(val_h, obs_h, feat_h, tab_obs_h, tab_feat_h, w_h, bias_h,
              out_h, tab_o, tab_f, w_v, b_v, obs_v, feat_v, val_v, out_v,
              sem_in0, sem_in1, sem_out0, sem_out1):
        wid = lax.axis_index("s") * NUM_CORES + lax.axis_index("c")
        base = wid * per_w
        sems_in = (sem_in0, sem_in1)
        sems_out = (sem_out0, sem_out1)

        pltpu.sync_copy(tab_obs_h, tab_o)
        pltpu.sync_copy(tab_feat_h, tab_f)
        pltpu.sync_copy(w_h, w_v)
        pltpu.sync_copy(bias_h, b_v)

        w_regs = [w_v[pl.ds(16 * j, 16)] for j in range(NJ)]
        b_regs = [b_v[pl.ds(16 * j, 16)] for j in range(NJ)]

        def in_copies(s, b):
            row0 = base + s * BLK
            return (
                pltpu.make_async_copy(obs_h.at[pl.ds(row0, BLK)],
                                      obs_v.at[pl.ds(b * BLK, BLK)],
                                      sems_in[b]),
                pltpu.make_async_copy(feat_h.at[pl.ds(row0, BLK)],
                                      feat_v.at[pl.ds(b * BLK, BLK)],
                                      sems_in[b]),
                pltpu.make_async_copy(val_h.at[pl.ds(row0, BLK)],
                                      val_v.at[pl.ds(b * BLK, BLK)],
                                      sems_in[b]),
            )

        def out_copy(s, b):
            row0 = base + s * BLK
            return pltpu.make_async_copy(
                out_v.at[pl.ds(b * BLK * D_MODEL, BLK * D_MODEL)],
                out_h.at[pl.ds(row0 * D_MODEL, BLK * D_MODEL)],
                sems_out[b])

        for c in in_copies(0, 0):
            c.start()

        def pair_body(g, carry):
            for b in range(2):
                s = g * 2 + b

                @pl.when(s + 1 < steps)
                def _():
                    for c in in_copies(s + 1, 1 - b):
                        c.start()

                for c in in_copies(s, b):
                    c.wait()

                @pl.when(s >= 2)
                def _():
                    out_copy(s - 2, b).wait()

                def grp_body(gi, carry2):
                    t0 = gi * 16
                    o16 = obs_v[pl.ds(b * BLK + t0, 16)]
                    f16 = feat_v[pl.ds(b * BLK + t0, 16)]
                    v16 = val_v[pl.ds(b * BLK + t0, 16)]
                    for k in range(16):
                        o = o16[k]
                        f = f16[k]
                        v = v16[k]
                        obase = o * D_MODEL
                        fbase = f * D_MODEL
                        dst = (b * BLK + t0 + k) * D_MODEL
                        for j in range(NJ):
                            ro = tab_o[pl.ds(obase + 16 * j, 16)]
                            rf = tab_f[pl.ds(fbase + 16 * j, 16)]
                            out_v[pl.ds(dst + 16 * j, 16)] = (
                                ro + rf + v * w_regs[j] + b_regs[j])
                    return carry2

                lax.fori_loop(0, BLK // 16, grp_body, 0)
                out_copy(s, b).start()
            return carry

        lax.fori_loop(0, steps // 2, pair_body, 0)
        out_copy(steps - 2, 0).wait()
        out_copy(steps - 1, 1).wait()

    return embed


def kernel(val, obs_idx, feat_idx, W_val, b_val, emb_obs, emb_feat):
    B, L, _ = val.shape
    T = B * L
    val_f = val.reshape(T).astype(jnp.float32)
    obs_f = obs_idx.reshape(T).astype(jnp.int32)
    feat_f = feat_idx.reshape(T).astype(jnp.int32)
    w_f = W_val.reshape(D_MODEL).astype(jnp.float32)
    b_f = b_val.reshape(D_MODEL).astype(jnp.float32)
    out = _build(T)(val_f, obs_f, feat_f,
                    emb_obs.astype(jnp.float32).reshape(N_OBS * D_MODEL),
                    emb_feat.astype(jnp.float32).reshape(N_FEAT * D_MODEL),
                    w_f, b_f)
    return out.reshape(B, L, D_MODEL)
